# baseline (speedup 1.0000x reference)
import functools

import jax
import jax.numpy as jnp
from jax import lax
from jax.experimental import pallas as pl
from jax.experimental.pallas import tpu as pltpu
from jax.experimental.pallas import tpu_sc as plsc

EMBED = 64
SCALE = 8.0
NC = 2
NS = 16
NW = NC * NS
GROUP = 128
NBUF = 4


@functools.lru_cache(maxsize=None)
def _make_gather(num_tokens):
    assert num_tokens % (NW * GROUP) == 0
    g_per_w = num_tokens // (NW * GROUP)
    assert g_per_w % NBUF == 0
    steps = g_per_w // NBUF
    mesh = plsc.VectorSubcoreMesh(core_axis_name="c", subcore_axis_name="s")

    @functools.partial(
        pl.kernel,
        mesh=mesh,
        out_type=jax.ShapeDtypeStruct((num_tokens, EMBED), jnp.float32),
        scratch_types=(
            [pltpu.VMEM((g_per_w, GROUP), jnp.int32)]
            + [pltpu.VMEM((GROUP, EMBED), jnp.float32) for _ in range(NBUF)]
            + [pltpu.SemaphoreType.DMA for _ in range(2 * NBUF)]
        ),
        compiler_params=pltpu.CompilerParams(use_tc_tiling_on_sc=False),
    )
    def gather(tok_hbm, tab_hbm, out_hbm, idx_v, *rest):
        rows = rest[:NBUF]
        sg = rest[NBUF:2 * NBUF]
        sw = rest[2 * NBUF:]
        wid = lax.axis_index("s") * NC + lax.axis_index("c")
        base = wid * (g_per_w * GROUP)
        pltpu.sync_copy(tok_hbm.at[wid], idx_v)

        def step(p, carry):
            g0 = p * NBUF
            for b in range(NBUF):
                pltpu.make_async_copy(
                    tab_hbm.at[idx_v.at[g0 + b]], rows[b], sg[b]).start()
            for b in range(NBUF):
                pltpu.make_async_copy(
                    tab_hbm.at[idx_v.at[g0 + b]], rows[b], sg[b]).wait()
                pltpu.make_async_copy(
                    rows[b],
                    out_hbm.at[pl.ds(base + (g0 + b) * GROUP, GROUP)],
                    sw[b]).start()
            for b in range(NBUF):
                pltpu.make_async_copy(
                    rows[b],
                    out_hbm.at[pl.ds(base + (g0 + b) * GROUP, GROUP)],
                    sw[b]).wait()
            return carry

        lax.fori_loop(0, steps, step, 0)

    return gather


def _unpack_scale(packed, num_tokens):
    rows_in = 1600
    nblk = num_tokens // (2 * rows_in)

    def body(p_hbm, o_ref, v0, v1, sems):
        i = pl.program_id(0)

        def dma(j, b):
            buf = v0 if b == 0 else v1
            return pltpu.make_async_copy(
                p_hbm.at[pl.ds(j * rows_in, rows_in)], buf, sems.at[b])

        @pl.when(i == 0)
        def _():
            dma(0, 0).start()
            dma(0, 0).wait()
            dma(1, 1).start()

        @pl.when((i > 0) & (i % 2 == 0))
        def _():
            dma(i, 0).wait()
            dma(i + 1, 1).start()

        @pl.when(i % 2 == 1)
        def _():
            dma(i, 1).wait()

            @pl.when(i + 1 < nblk)
            def _():
                dma(i + 1, 0).start()

        x = jnp.where((i % 2) == 0, v0[...], v1[...]) * SCALE
        o_ref[pl.Slice(0, rows_in, 2), :] = x[:, :EMBED]
        o_ref[pl.Slice(1, rows_in, 2), :] = x[:, EMBED:]

    return pl.pallas_call(
        body,
        out_shape=jax.ShapeDtypeStruct((num_tokens, EMBED), jnp.float32),
        grid=(nblk,),
        in_specs=[pl.BlockSpec(memory_space=pl.ANY)],
        out_specs=pl.BlockSpec((2 * rows_in, EMBED), lambda i: (i, 0)),
        scratch_shapes=[
            pltpu.VMEM((rows_in, 2 * EMBED), jnp.float32),
            pltpu.VMEM((rows_in, 2 * EMBED), jnp.float32),
            pltpu.SemaphoreType.DMA((2,)),
        ],
    )(packed)


def kernel(tokens, table):
    batch, seq = tokens.shape
    num_tokens = batch * seq
    idx = tokens.astype(jnp.int32).reshape(NW, num_tokens // (NW * GROUP), GROUP)
    gathered = _make_gather(num_tokens)(idx, table)
    packed = gathered.reshape(num_tokens // 2, 2 * EMBED)
    return _unpack_scale(packed, num_tokens).reshape(batch, seq, EMBED)


# speedup vs baseline: 1.1074x; 1.1074x over previous
import functools

import jax
import jax.numpy as jnp
from jax import lax
from jax.experimental import pallas as pl
from jax.experimental.pallas import tpu as pltpu
from jax.experimental.pallas import tpu_sc as plsc

EMBED = 64
SCALE = 8.0
NC = 2
NS = 16
NW = NC * NS
GROUP = 128
NCHUNK = 4
NBUF = 5
ROWS_IN = 1600


@functools.lru_cache(maxsize=None)
def _make_gather(chunk_tokens):
    assert chunk_tokens % (NW * GROUP) == 0
    g_per_w = chunk_tokens // (NW * GROUP)
    assert g_per_w % NBUF == 0
    steps = g_per_w // NBUF
    mesh = plsc.VectorSubcoreMesh(core_axis_name="c", subcore_axis_name="s")

    @functools.partial(
        pl.kernel,
        mesh=mesh,
        out_type=jax.ShapeDtypeStruct((chunk_tokens, EMBED), jnp.float32),
        scratch_types=(
            [pltpu.VMEM((g_per_w, GROUP), jnp.int32)]
            + [pltpu.VMEM((GROUP, EMBED), jnp.float32) for _ in range(NBUF)]
            + [pltpu.SemaphoreType.DMA for _ in range(2 * NBUF)]
        ),
        compiler_params=pltpu.CompilerParams(use_tc_tiling_on_sc=False),
    )
    def gather(tok_hbm, tab_hbm, out_hbm, idx_v, *rest):
        rows = rest[:NBUF]
        sg = rest[NBUF:2 * NBUF]
        sw = rest[2 * NBUF:]
        wid = lax.axis_index("s") * NC + lax.axis_index("c")
        base = wid * (g_per_w * GROUP)
        pltpu.sync_copy(tok_hbm.at[wid], idx_v)

        def step(p, carry):
            g0 = p * NBUF
            for b in range(NBUF):
                pltpu.make_async_copy(
                    tab_hbm.at[idx_v.at[g0 + b]], rows[b], sg[b]).start()
            for b in range(NBUF):
                pltpu.make_async_copy(
                    tab_hbm.at[idx_v.at[g0 + b]], rows[b], sg[b]).wait()
                pltpu.make_async_copy(
                    rows[b],
                    out_hbm.at[pl.ds(base + (g0 + b) * GROUP, GROUP)],
                    sw[b]).start()
            for b in range(NBUF):
                pltpu.make_async_copy(
                    rows[b],
                    out_hbm.at[pl.ds(base + (g0 + b) * GROUP, GROUP)],
                    sw[b]).wait()
            return carry

        lax.fori_loop(0, steps, step, 0)

    return gather


def _out_block_index(i, chunk, per_w_blocks, chunk_blocks_per_w):
    w = i // chunk_blocks_per_w
    j = i % chunk_blocks_per_w
    return w * per_w_blocks + chunk * chunk_blocks_per_w + j


@functools.lru_cache(maxsize=None)
def _make_unpack(num_tokens, chunk_tokens, chunk):
    nblk = chunk_tokens // (2 * ROWS_IN)
    per_w_blocks = (num_tokens // NW) // (2 * ROWS_IN)
    chunk_blocks_per_w = nblk // NW

    def omap(i):
        return (_out_block_index(i, chunk, per_w_blocks, chunk_blocks_per_w), 0)

    def body(p_ref, *rest):
        o_ref = rest[-1]
        x = p_ref[...] * SCALE
        o_ref[pl.Slice(0, ROWS_IN, 2), :] = x[:, :EMBED]
        o_ref[pl.Slice(1, ROWS_IN, 2), :] = x[:, EMBED:]

    in_specs = [pl.BlockSpec((ROWS_IN, 2 * EMBED), lambda i: (i, 0))]
    aliases = {}
    if chunk > 0:
        in_specs.append(pl.BlockSpec(memory_space=pl.ANY))
        aliases = {1: 0}

    return pl.pallas_call(
        body,
        out_shape=jax.ShapeDtypeStruct((num_tokens, EMBED), jnp.float32),
        grid=(nblk,),
        in_specs=in_specs,
        out_specs=pl.BlockSpec((2 * ROWS_IN, EMBED), omap),
        input_output_aliases=aliases,
    )


def kernel(tokens, table):
    batch, seq = tokens.shape
    num_tokens = batch * seq
    chunk_tokens = num_tokens // NCHUNK
    g_per_w = num_tokens // (NW * GROUP)
    gc = g_per_w // NCHUNK
    idx = tokens.astype(jnp.int32).reshape(NW, g_per_w, GROUP)
    gather = _make_gather(chunk_tokens)
    out = None
    for c in range(NCHUNK):
        idx_c = lax.slice_in_dim(idx, c * gc, (c + 1) * gc, axis=1)
        packed = gather(idx_c, table).reshape(chunk_tokens // 2, 2 * EMBED)
        unpack = _make_unpack(num_tokens, chunk_tokens, c)
        out = unpack(packed) if c == 0 else unpack(packed, out)
    return out.reshape(batch, seq, EMBED)


# speedup vs baseline: 1.1097x; 1.0021x over previous
import functools

import jax
import jax.numpy as jnp
from jax import lax
from jax.experimental import pallas as pl
from jax.experimental.pallas import tpu as pltpu
from jax.experimental.pallas import tpu_sc as plsc

EMBED = 64
SCALE = 8.0
NC = 2
NS = 16
NW = NC * NS
GROUP = 128
NCHUNK = 8
NBUF = 5
ROWS_IN = 1600


@functools.lru_cache(maxsize=None)
def _make_gather(chunk_tokens):
    assert chunk_tokens % (NW * GROUP) == 0
    g_per_w = chunk_tokens // (NW * GROUP)
    assert g_per_w % NBUF == 0
    steps = g_per_w // NBUF
    mesh = plsc.VectorSubcoreMesh(core_axis_name="c", subcore_axis_name="s")

    @functools.partial(
        pl.kernel,
        mesh=mesh,
        out_type=jax.ShapeDtypeStruct((chunk_tokens, EMBED), jnp.float32),
        scratch_types=(
            [pltpu.VMEM((g_per_w, GROUP), jnp.int32)]
            + [pltpu.VMEM((GROUP, EMBED), jnp.float32) for _ in range(NBUF)]
            + [pltpu.SemaphoreType.DMA for _ in range(2 * NBUF)]
        ),
        compiler_params=pltpu.CompilerParams(use_tc_tiling_on_sc=False),
    )
    def gather(tok_hbm, tab_hbm, out_hbm, idx_v, *rest):
        rows = rest[:NBUF]
        sg = rest[NBUF:2 * NBUF]
        sw = rest[2 * NBUF:]
        wid = lax.axis_index("s") * NC + lax.axis_index("c")
        base = wid * (g_per_w * GROUP)
        pltpu.sync_copy(tok_hbm.at[wid], idx_v)

        def step(p, carry):
            g0 = p * NBUF
            for b in range(NBUF):
                pltpu.make_async_copy(
                    tab_hbm.at[idx_v.at[g0 + b]], rows[b], sg[b]).start()
            for b in range(NBUF):
                pltpu.make_async_copy(
                    tab_hbm.at[idx_v.at[g0 + b]], rows[b], sg[b]).wait()
                pltpu.make_async_copy(
                    rows[b],
                    out_hbm.at[pl.ds(base + (g0 + b) * GROUP, GROUP)],
                    sw[b]).start()
            for b in range(NBUF):
                pltpu.make_async_copy(
                    rows[b],
                    out_hbm.at[pl.ds(base + (g0 + b) * GROUP, GROUP)],
                    sw[b]).wait()
            return carry

        lax.fori_loop(0, steps, step, 0)

    return gather


def _out_block_index(i, chunk, per_w_blocks, chunk_blocks_per_w):
    w = i // chunk_blocks_per_w
    j = i % chunk_blocks_per_w
    return w * per_w_blocks + chunk * chunk_blocks_per_w + j


@functools.lru_cache(maxsize=None)
def _make_unpack(num_tokens, chunk_tokens, chunk):
    nblk = chunk_tokens // (2 * ROWS_IN)
    per_w_blocks = (num_tokens // NW) // (2 * ROWS_IN)
    chunk_blocks_per_w = nblk // NW

    def omap(i):
        return (_out_block_index(i, chunk, per_w_blocks, chunk_blocks_per_w), 0)

    def body(p_ref, *rest):
        o_ref = rest[-1]
        x = p_ref[...] * SCALE
        o_ref[pl.Slice(0, ROWS_IN, 2), :] = x[:, :EMBED]
        o_ref[pl.Slice(1, ROWS_IN, 2), :] = x[:, EMBED:]

    in_specs = [pl.BlockSpec((ROWS_IN, 2 * EMBED), lambda i: (i, 0))]
    aliases = {}
    if chunk > 0:
        in_specs.append(pl.BlockSpec(memory_space=pl.ANY))
        aliases = {1: 0}

    return pl.pallas_call(
        body,
        out_shape=jax.ShapeDtypeStruct((num_tokens, EMBED), jnp.float32),
        grid=(nblk,),
        in_specs=in_specs,
        out_specs=pl.BlockSpec((2 * ROWS_IN, EMBED), omap),
        input_output_aliases=aliases,
    )


def kernel(tokens, table):
    batch, seq = tokens.shape
    num_tokens = batch * seq
    chunk_tokens = num_tokens // NCHUNK
    g_per_w = num_tokens // (NW * GROUP)
    gc = g_per_w // NCHUNK
    idx = tokens.astype(jnp.int32).reshape(NW, g_per_w, GROUP)
    gather = _make_gather(chunk_tokens)
    out = None
    for c in range(NCHUNK):
        idx_c = lax.slice_in_dim(idx, c * gc, (c + 1) * gc, axis=1)
        packed = gather(idx_c, table).reshape(chunk_tokens // 2, 2 * EMBED)
        unpack = _make_unpack(num_tokens, chunk_tokens, c)
        out = unpack(packed) if c == 0 else unpack(packed, out)
    return out.reshape(batch, seq, EMBED)
